# baseline (speedup 1.0000x reference)
import functools
import numpy as np
import jax
import jax.numpy as jnp
from jax import lax
from jax.experimental import pallas as pl
from jax.experimental.pallas import tpu as pltpu
from jax.experimental.pallas import tpu_sc as plsc

_B, _C, _F, _T = 16, 1, 128, 4096
_L = 16
_NC, _NS = 2, 16
_NW = _NC * _NS
_ROWS = _B * _F
_RPW = _ROWS // _NW
_CH = 4
_NCHUNK = _RPW // _CH
_TG = _T // _L
_TT = _T // 128
_TR = _ROWS // 8

_factors_f64 = np.linspace(1.0, 3.0, _B)
_ti = np.arange(_T)[None, :] / _factors_f64[:, None]
_floor_np = np.floor(_ti).astype(np.int32).reshape(-1)
_frac_np = (_ti - np.floor(_ti)).astype(np.float32).reshape(-1)
_factors_np = _factors_f64.astype(np.float32)

_mesh = plsc.VectorSubcoreMesh(core_axis_name="c", subcore_axis_name="s")


@functools.partial(
    pl.kernel,
    out_type=(
        jax.ShapeDtypeStruct((_TR, _TT, 8, 128), jnp.float32),
        jax.ShapeDtypeStruct((_B,), jnp.int32),
    ),
    mesh=_mesh,
    compiler_params=pltpu.CompilerParams(needs_layout_passes=False),
    scratch_types=[
        pltpu.VMEM((_T,), jnp.int32),
        pltpu.VMEM((_T,), jnp.float32),
        pltpu.VMEM((_CH, _TT, 128), jnp.float32),
        pltpu.VMEM((_CH, _TT, 128), jnp.float32),
        pltpu.VMEM((_CH, _TT, 128), jnp.float32),
        pltpu.VMEM((_CH, _TT, 128), jnp.float32),
        pltpu.VMEM((_B,), jnp.int32),
        pltpu.VMEM((_B,), jnp.float32),
        pltpu.VMEM((_B,), jnp.int32),
        pltpu.SemaphoreType.DMA,
        pltpu.SemaphoreType.DMA,
        pltpu.SemaphoreType.DMA,
        pltpu.SemaphoreType.DMA,
    ],
)
def _warp_kernel(x_hbm, seqlen_hbm, floor_hbm, frac_hbm, fac_hbm,
                 out_hbm, nsl_hbm,
                 floor_v, frac_v, xin0, xin1, xout0, xout1,
                 seq_v, fac_v, nsl_v,
                 isem0, isem1, osem0, osem1):
    wid = lax.axis_index("s") * _NC + lax.axis_index("c")
    b = wid // 2
    row0 = b * _F + (wid % 2) * _RPW
    xin = (xin0, xin1)
    xout = (xout0, xout1)
    isem = (isem0, isem1)
    osem = (osem0, osem1)

    pltpu.sync_copy(fac_hbm, fac_v)

    @pl.when(wid == 0)
    def _():
        pltpu.sync_copy(seqlen_hbm, seq_v)
        s = seq_v[...].astype(jnp.float32) * fac_v[...]
        nsl_v[...] = jnp.minimum(s, jnp.float32(_T)).astype(jnp.int32)
        pltpu.sync_copy(nsl_v, nsl_hbm)

    pltpu.sync_copy(floor_hbm.at[pl.ds(b * _T, _T)], floor_v)
    pltpu.sync_copy(frac_hbm.at[pl.ds(b * _T, _T)], frac_v)

    def in_start(ch, par):
        r0 = row0 + ch * _CH
        for r in range(_CH):
            row = r0 + r
            pltpu.async_copy(
                x_hbm.at[row // 8, :, row % 8, :], xin[par].at[r], isem[par])

    def in_wait(par):
        for r in range(_CH):
            pltpu.make_async_copy(
                x_hbm.at[0, :, 0, :], xin[par].at[r], isem[par]).wait()

    def out_start(ch, par):
        r0 = row0 + ch * _CH
        for r in range(_CH):
            row = r0 + r
            pltpu.async_copy(
                xout[par].at[r], out_hbm.at[row // 8, :, row % 8, :], osem[par])

    def out_wait(par):
        for r in range(_CH):
            pltpu.make_async_copy(
                xout[par].at[r], out_hbm.at[0, :, 0, :], osem[par]).wait()

    in_start(0, 0)
    in_start(1, 1)

    def chunk_pair(k, _):
        for par in (0, 1):
            ch = 2 * k + par
            in_wait(par)

            @pl.when(k > 0)
            def _():
                out_wait(par)

            src = xin[par]
            dst = xout[par]

            @plsc.parallel_loop(0, _TG, unroll=8)
            def _(i):
                off = i * _L
                fi = floor_v[pl.ds(off, _L)]
                fr = frac_v[pl.ds(off, _L)]
                fi1 = jnp.minimum(fi + 1, _T - 1)
                fi_hi = lax.shift_right_logical(fi, 7)
                fi_lo = lax.bitwise_and(fi, 127)
                fi1_hi = lax.shift_right_logical(fi1, 7)
                fi1_lo = lax.bitwise_and(fi1, 127)
                tc = i // 8
                lo = (i % 8) * _L
                for r in range(_CH):
                    ridx = jnp.full((_L,), r, jnp.int32)
                    a = plsc.load_gather(src, [ridx, fi_hi, fi_lo])
                    c = plsc.load_gather(src, [ridx, fi1_hi, fi1_lo])
                    dst[r, tc, pl.ds(lo, _L)] = a + fr * (c - a)

            out_start(ch, par)

            @pl.when(k < (_NCHUNK // 2) - 1)
            def _():
                in_start(ch + 2, par)
        return 0

    lax.fori_loop(0, _NCHUNK // 2, chunk_pair, 0)
    out_wait(0)
    out_wait(1)


def kernel(x, seq_len):
    xt = x.reshape(_TR, 8, _TT, 128).swapaxes(1, 2)
    out_t, new_seq_len = _warp_kernel(
        xt, seq_len,
        jnp.asarray(_floor_np), jnp.asarray(_frac_np), jnp.asarray(_factors_np),
    )
    out = out_t.swapaxes(1, 2).reshape(_B, _C, _F, _T)
    return out, new_seq_len


# speedup vs baseline: 1.0618x; 1.0618x over previous
import functools
import numpy as np
import jax
import jax.numpy as jnp
from jax import lax
from jax.experimental import pallas as pl
from jax.experimental.pallas import tpu as pltpu
from jax.experimental.pallas import tpu_sc as plsc

_B, _C, _F, _T = 16, 1, 128, 4096
_L = 16
_NC, _NS = 2, 16
_NW = _NC * _NS
_ROWS = _B * _F
_RPW = _ROWS // _NW
_CH = 4
_NCHUNK = _RPW // _CH
_TG = _T // _L
_TT = _T // 128
_TR = _ROWS // 8

_factors_f64 = np.linspace(1.0, 3.0, _B)
_ti = np.arange(_T)[None, :] / _factors_f64[:, None]
_ti_np = _ti.astype(np.float32).reshape(-1)
_factors_np = _factors_f64.astype(np.float32)

_mesh = plsc.VectorSubcoreMesh(core_axis_name="c", subcore_axis_name="s")


@functools.partial(
    pl.kernel,
    out_type=(
        jax.ShapeDtypeStruct((_TR, _TT, 8, 128), jnp.float32),
        jax.ShapeDtypeStruct((_B,), jnp.int32),
    ),
    mesh=_mesh,
    compiler_params=pltpu.CompilerParams(needs_layout_passes=False),
    scratch_types=[
        pltpu.VMEM((_T,), jnp.float32),
        pltpu.VMEM((_CH, _TT, 128), jnp.float32),
        pltpu.VMEM((_CH, _TT, 128), jnp.float32),
        pltpu.VMEM((_CH, _TT, 128), jnp.float32),
        pltpu.VMEM((_CH, _TT, 128), jnp.float32),
        pltpu.VMEM((_B,), jnp.int32),
        pltpu.VMEM((_B,), jnp.float32),
        pltpu.VMEM((_B,), jnp.int32),
        pltpu.SemaphoreType.DMA,
        pltpu.SemaphoreType.DMA,
        pltpu.SemaphoreType.DMA,
        pltpu.SemaphoreType.DMA,
    ],
)
def _warp_kernel(x_hbm, seqlen_hbm, ti_hbm, fac_hbm,
                 out_hbm, nsl_hbm,
                 ti_v, xin0, xin1, xout0, xout1,
                 seq_v, fac_v, nsl_v,
                 isem0, isem1, osem0, osem1):
    wid = lax.axis_index("s") * _NC + lax.axis_index("c")
    b = wid // 2
    row0 = b * _F + (wid % 2) * _RPW
    xin = (xin0, xin1)
    xout = (xout0, xout1)
    isem = (isem0, isem1)
    osem = (osem0, osem1)

    pltpu.sync_copy(fac_hbm, fac_v)

    @pl.when(wid == 0)
    def _():
        pltpu.sync_copy(seqlen_hbm, seq_v)
        s = seq_v[...].astype(jnp.float32) * fac_v[...]
        nsl_v[...] = jnp.minimum(s, jnp.float32(_T)).astype(jnp.int32)
        pltpu.sync_copy(nsl_v, nsl_hbm)

    pltpu.sync_copy(ti_hbm.at[pl.ds(b * _T, _T)], ti_v)

    def in_start(ch, par):
        r0 = row0 + ch * _CH
        for r in range(_CH):
            row = r0 + r
            pltpu.async_copy(
                x_hbm.at[row // 8, :, row % 8, :], xin[par].at[r], isem[par])

    def in_wait(par):
        for r in range(_CH):
            pltpu.make_async_copy(
                x_hbm.at[0, :, 0, :], xin[par].at[r], isem[par]).wait()

    def out_start(ch, par):
        r0 = row0 + ch * _CH
        for r in range(_CH):
            row = r0 + r
            pltpu.async_copy(
                xout[par].at[r], out_hbm.at[row // 8, :, row % 8, :], osem[par])

    def out_wait(par):
        for r in range(_CH):
            pltpu.make_async_copy(
                xout[par].at[r], out_hbm.at[0, :, 0, :], osem[par]).wait()

    in_start(0, 0)
    in_start(1, 1)

    def chunk_pair(k, _):
        for par in (0, 1):
            ch = 2 * k + par
            in_wait(par)

            @pl.when(k > 0)
            def _():
                out_wait(par)

            src = xin[par]
            dst = xout[par]

            @plsc.parallel_loop(0, _TG, unroll=4)
            def _(i):
                off = i * _L
                tiv = ti_v[pl.ds(off, _L)]
                fi = tiv.astype(jnp.int32)
                fr = tiv - fi.astype(jnp.float32)
                fi1 = jnp.minimum(fi + 1, _T - 1)
                fi_hi = lax.shift_right_logical(fi, 7)
                fi_lo = lax.bitwise_and(fi, 127)
                fi1_hi = lax.shift_right_logical(fi1, 7)
                fi1_lo = lax.bitwise_and(fi1, 127)
                tc = i // 8
                lo = (i % 8) * _L
                for r in range(_CH):
                    ridx = jnp.full((_L,), r, jnp.int32)
                    a = plsc.load_gather(src, [ridx, fi_hi, fi_lo])
                    c = plsc.load_gather(src, [ridx, fi1_hi, fi1_lo])
                    dst[r, tc, pl.ds(lo, _L)] = a + fr * (c - a)

            out_start(ch, par)

            @pl.when(k < (_NCHUNK // 2) - 1)
            def _():
                in_start(ch + 2, par)
        return 0

    lax.fori_loop(0, _NCHUNK // 2, chunk_pair, 0)
    out_wait(0)
    out_wait(1)


def kernel(x, seq_len):
    xt = x.reshape(_TR, 8, _TT, 128).swapaxes(1, 2)
    out_t, new_seq_len = _warp_kernel(
        xt, seq_len,
        jnp.asarray(_ti_np), jnp.asarray(_factors_np),
    )
    out = out_t.swapaxes(1, 2).reshape(_B, _C, _F, _T)
    return out, new_seq_len
